# baseline (speedup 1.0000x reference)
import functools

import jax
import jax.numpy as jnp
from jax import lax
from jax.experimental import pallas as pl
from jax.experimental.pallas import tpu as pltpu
from jax.experimental.pallas import tpu_sc as plsc

N = 10000
E = 160000
H = 600
HP = 640
F = 128
NCH = 5
KB = 80
TILES = 16
BPT = (E // TILES) // KB
SLAB = N // TILES
ZR = 25
RT = 400
NT = N // RT
EPS = 1e-5


def _sc_segsum(hflat, src2, dst2):
    mesh = plsc.VectorSubcoreMesh(core_axis_name="c", subcore_axis_name="s")

    @functools.partial(
        pl.kernel,
        mesh=mesh,
        compiler_params=pltpu.CompilerParams(use_tc_tiling_on_sc=False),
        out_type=jax.ShapeDtypeStruct(((NCH + 1) * N, F), jnp.float32),
        scratch_types=[
            pltpu.VMEM((BPT, KB), jnp.int32),
            pltpu.VMEM((BPT, KB), jnp.int32),
            pltpu.VMEM((KB, F), jnp.float32),
            pltpu.VMEM((KB, F), jnp.float32),
            pltpu.VMEM((ZR, F), jnp.float32),
            pltpu.VMEM_SHARED((N, F), jnp.float32),
            pltpu.SemaphoreType.DMA,
            pltpu.SemaphoreType.DMA,
        ],
    )
    def k(h_hbm, src_hbm, dst_hbm, m_hbm, srcv, dstv, st0, st1, zbuf, acc,
          sem0, sem1):
        cc = lax.axis_index("c")
        ss = lax.axis_index("s")
        row0 = ss * BPT
        pltpu.sync_copy(src_hbm.at[pl.ds(row0, BPT)], srcv)
        pltpu.sync_copy(dst_hbm.at[pl.ds(row0, BPT)], dstv)

        def zrow(r, _):
            for kk in range(F // 16):
                zbuf[r, pl.ds(kk * 16, 16)] = jnp.zeros((16,), jnp.float32)
            return 0

        lax.fori_loop(0, ZR, zrow, 0)

        for j in range(3):
            c = 3 * cc + j
            if True:
                if j == 0:
                    delta = (3 * cc * N).astype(jnp.int32)
                elif j == 1:
                    delta = jnp.int32(N)
                else:
                    delta = jnp.where(cc == 0, N, -2 * N).astype(jnp.int32)

                def addrow(r, _):
                    for kk in range(KB // 16):
                        sl = pl.ds(kk * 16, 16)
                        srcv[r, sl] = srcv[r, sl] + delta
                    return 0

                lax.fori_loop(0, BPT, addrow, 0)

                for q in range(SLAB // ZR):
                    pltpu.sync_copy(
                        zbuf, acc.at[pl.ds(ss * SLAB + q * ZR, ZR)])
                plsc.subcore_barrier()

                if j < 2:
                    pltpu.async_copy(h_hbm.at[srcv.at[0]], st0, sem0)

                    def eblk(k2, _):
                        b0 = 2 * k2
                        b1 = b0 + 1
                        pltpu.async_copy(h_hbm.at[srcv.at[b1]], st1, sem1)
                        pltpu.make_async_copy(
                            h_hbm.at[srcv.at[b0]], st0, sem0).wait()
                        pltpu.sync_copy(st0, acc.at[dstv.at[b0]], add=True)

                        @pl.when(b0 + 2 < BPT)
                        def _():
                            pltpu.async_copy(
                                h_hbm.at[srcv.at[b0 + 2]], st0, sem0)

                        pltpu.make_async_copy(
                            h_hbm.at[srcv.at[b1]], st1, sem1).wait()
                        pltpu.sync_copy(st1, acc.at[dstv.at[b1]], add=True)
                        return 0

                    lax.fori_loop(0, (BPT - 1) // 2, eblk, 0)
                    pltpu.make_async_copy(
                        h_hbm.at[srcv.at[BPT - 1]], st0, sem0).wait()
                    pltpu.sync_copy(st0, acc.at[dstv.at[BPT - 1]], add=True)
                else:
                    lo = 62 * cc

                    pltpu.async_copy(h_hbm.at[srcv.at[lo]], st0, sem0)

                    def eblk2(k2, _):
                        b0 = lo + 2 * k2
                        b1 = b0 + 1
                        pltpu.async_copy(h_hbm.at[srcv.at[b1]], st1, sem1)
                        pltpu.make_async_copy(
                            h_hbm.at[srcv.at[b0]], st0, sem0).wait()
                        pltpu.sync_copy(st0, acc.at[dstv.at[b0]], add=True)

                        @pl.when(k2 + 1 < 31)
                        def _():
                            pltpu.async_copy(
                                h_hbm.at[srcv.at[b0 + 2]], st0, sem0)

                        pltpu.make_async_copy(
                            h_hbm.at[srcv.at[b1]], st1, sem1).wait()
                        pltpu.sync_copy(st1, acc.at[dstv.at[b1]], add=True)
                        return 0

                    lax.fori_loop(0, 31, eblk2, 0)

                    @pl.when(cc == 0)
                    def _():
                        pltpu.sync_copy(h_hbm.at[srcv.at[BPT - 1]], st0)
                        pltpu.sync_copy(
                            st0, acc.at[dstv.at[BPT - 1]], add=True)
                plsc.subcore_barrier()

                pltpu.sync_copy(
                    acc.at[pl.ds(ss * SLAB, SLAB)],
                    m_hbm.at[pl.ds(c * N + ss * SLAB, SLAB)],
                )
                plsc.subcore_barrier()

    return k(hflat, src2, dst2)


def _mm_stats(h4, m4, w1, b1, first_layer):
    kw = HP if first_layer else H

    def body(h_ref, m_ref, w_ref, b_ref, t_ref, s_ref, stats):
        i = pl.program_id(0)
        hm = h_ref[...]
        mm = m_ref[...]
        parts = []
        for cch in range(NCH):
            mc = mm[cch] + mm[NCH] if cch == 2 else mm[cch]
            parts.append(hm[cch] + mc)
        ut = jnp.concatenate(parts, axis=1)
        if first_layer:
            t = jnp.dot(ut, w_ref[...], preferred_element_type=jnp.float32)
            t = t[:, :H]
        else:
            t = jnp.dot(ut[:, :H], w_ref[...],
                        preferred_element_type=jnp.float32)
        t = t + b_ref[...]
        t_ref[...] = t

        @pl.when(i == 0)
        def _():
            stats[...] = jnp.zeros_like(stats)

        stats[0:1] = stats[0:1] + jnp.sum(t, axis=0, keepdims=True)
        stats[1:2] = stats[1:2] + jnp.sum(t * t, axis=0, keepdims=True)
        s_ref[...] = stats[...]

    return pl.pallas_call(
        body,
        grid=(NT,),
        in_specs=[
            pl.BlockSpec((NCH, RT, F), lambda i: (0, i, 0)),
            pl.BlockSpec((NCH + 1, RT, F), lambda i: (0, i, 0)),
            pl.BlockSpec((kw, H), lambda i: (0, 0)),
            pl.BlockSpec((1, H), lambda i: (0, 0)),
        ],
        out_specs=[
            pl.BlockSpec((RT, H), lambda i: (i, 0)),
            pl.BlockSpec((8, H), lambda i: (0, 0)),
        ],
        out_shape=[
            jax.ShapeDtypeStruct((N, H), jnp.float32),
            jax.ShapeDtypeStruct((8, H), jnp.float32),
        ],
        scratch_shapes=[pltpu.VMEM((8, H), jnp.float32)],
    )(h4, m4, w1, b1)


def _fused_layer(h4, m4, w1, b1, g1, be1, w2, b2, first_layer, relu_out):
    kw = HP if first_layer else H

    def body(h_ref, m_ref, w1_ref, b1_ref, g_ref, be_ref, w2_ref, b2_ref,
             out_ref, tbuf, stats):
        p = pl.program_id(0)
        i = pl.program_id(1)

        @pl.when(p == 0)
        def _():
            hm = h_ref[...]
            mm = m_ref[...]
            parts = []
            for cch in range(NCH):
                mc = mm[cch] + mm[NCH] if cch == 2 else mm[cch]
                parts.append(hm[cch] + mc)
            ut = jnp.concatenate(parts, axis=1)
            if first_layer:
                t = jnp.dot(ut, w1_ref[...],
                            preferred_element_type=jnp.float32)[:, :H]
            else:
                t = jnp.dot(ut[:, :H], w1_ref[...],
                            preferred_element_type=jnp.float32)
            t = t + b1_ref[...]
            tbuf[pl.ds(i * RT, RT), :] = t

            @pl.when(i == 0)
            def _():
                stats[...] = jnp.zeros_like(stats)

            stats[0:1] = stats[0:1] + jnp.sum(t, axis=0, keepdims=True)
            stats[1:2] = stats[1:2] + jnp.sum(t * t, axis=0, keepdims=True)

        @pl.when(p == 1)
        def _():
            mean = stats[0:1] / N
            var = stats[1:2] / N - mean * mean
            scale = g_ref[...] * lax.rsqrt(var + EPS)
            shift = be_ref[...] - mean * scale
            t = tbuf[pl.ds(i * RT, RT), :]
            z = jnp.maximum(t * scale + shift, 0.0)
            o = jnp.dot(z, w2_ref[...], preferred_element_type=jnp.float32)
            o = o + b2_ref[...]
            if relu_out:
                o = jnp.maximum(o, 0.0)
            o = jnp.concatenate(
                [o, jnp.zeros((RT, HP - H), jnp.float32)], axis=1)
            for cch in range(NCH):
                out_ref[cch] = o[:, cch * F:(cch + 1) * F]

    return pl.pallas_call(
        body,
        grid=(2, NT),
        in_specs=[
            pl.BlockSpec((NCH, RT, F), lambda p, i: (0, i, 0)),
            pl.BlockSpec((NCH + 1, RT, F), lambda p, i: (0, i, 0)),
            pl.BlockSpec((kw, H), lambda p, i: (0, 0)),
            pl.BlockSpec((1, H), lambda p, i: (0, 0)),
            pl.BlockSpec((1, H), lambda p, i: (0, 0)),
            pl.BlockSpec((1, H), lambda p, i: (0, 0)),
            pl.BlockSpec((H, H), lambda p, i: (0, 0)),
            pl.BlockSpec((1, H), lambda p, i: (0, 0)),
        ],
        out_specs=pl.BlockSpec((NCH, RT, F), lambda p, i: (0, i, 0)),
        out_shape=jax.ShapeDtypeStruct((NCH, N, F), jnp.float32),
        scratch_shapes=[pltpu.VMEM((N, H), jnp.float32),
                        pltpu.VMEM((8, H), jnp.float32)],
    )(h4, m4, w1, b1, g1, be1, w2, b2)


def _bn_relu_mm(t, stats, g1, be1, w2, b2, relu_out):
    def body(t_ref, s_ref, g_ref, be_ref, w_ref, b_ref, out_ref):
        mean = s_ref[0:1] / N
        var = s_ref[1:2] / N - mean * mean
        scale = g_ref[...] * lax.rsqrt(var + EPS)
        shift = be_ref[...] - mean * scale
        z = jnp.maximum(t_ref[...] * scale + shift, 0.0)
        o = jnp.dot(z, w_ref[...], preferred_element_type=jnp.float32)
        o = o + b_ref[...]
        if relu_out:
            o = jnp.maximum(o, 0.0)
        o = jnp.concatenate([o, jnp.zeros((RT, HP - H), jnp.float32)], axis=1)
        for cch in range(NCH):
            out_ref[cch] = o[:, cch * F:(cch + 1) * F]

    return pl.pallas_call(
        body,
        grid=(NT,),
        in_specs=[
            pl.BlockSpec((RT, H), lambda i: (i, 0)),
            pl.BlockSpec((8, H), lambda i: (0, 0)),
            pl.BlockSpec((1, H), lambda i: (0, 0)),
            pl.BlockSpec((1, H), lambda i: (0, 0)),
            pl.BlockSpec((H, H), lambda i: (0, 0)),
            pl.BlockSpec((1, H), lambda i: (0, 0)),
        ],
        out_specs=pl.BlockSpec((NCH, RT, F), lambda i: (0, i, 0)),
        out_shape=jax.ShapeDtypeStruct((NCH, N, F), jnp.float32),
    )(t, stats, g1, be1, w2, b2)


def _pool_classifier(h4, batch3, w1, b1, g1, be1, w2, b2, g2, be2, w3, b3,
                     num_tasks, g_graphs):
    def bn(y, g, be):
        mu = jnp.mean(y, axis=0, keepdims=True)
        var = jnp.mean(y * y, axis=0, keepdims=True) - mu * mu
        return (y - mu) * lax.rsqrt(var + EPS) * g + be

    def body(h_ref, b_ref, w1_ref, b1_ref, g1_ref, be1_ref, w2_ref, b2_ref,
             g2_ref, be2_ref, w3_ref, b3_ref, out_ref, feats):
        t = pl.program_id(0)

        @pl.when(t == 0)
        def _():
            feats[...] = jnp.zeros_like(feats)

        @pl.when(t < NT)
        def _():
            ht = h_ref[...].transpose(1, 0, 2).reshape(RT, HP)[:, :H]
            gids = b_ref[0, 0, :]
            rows = lax.broadcasted_iota(jnp.int32, (g_graphs, RT), 0)
            oh = (rows == gids[None, :]).astype(jnp.float32)
            feats[...] = feats[...] + jnp.dot(
                oh, ht, preferred_element_type=jnp.float32,
                precision=lax.Precision.HIGHEST)

        @pl.when(t == NT)
        def _():
            f = feats[...]
            y = jnp.dot(f, w1_ref[...], preferred_element_type=jnp.float32)
            y = jnp.maximum(bn(y + b1_ref[...], g1_ref[...], be1_ref[...]), 0.0)
            y = jnp.dot(y, w2_ref[...], preferred_element_type=jnp.float32)
            y = jnp.maximum(bn(y + b2_ref[...], g2_ref[...], be2_ref[...]), 0.0)
            y = jnp.dot(y, w3_ref[...], preferred_element_type=jnp.float32)
            out_ref[...] = y + b3_ref[...]

    def clamp(t):
        return jnp.minimum(t, NT - 1)

    return pl.pallas_call(
        body,
        grid=(NT + 1,),
        in_specs=[
            pl.BlockSpec((NCH, RT, F), lambda t: (0, clamp(t), 0)),
            pl.BlockSpec((1, 1, RT), lambda t: (clamp(t), 0, 0)),
            pl.BlockSpec((H, 256), lambda t: (0, 0)),
            pl.BlockSpec((1, 256), lambda t: (0, 0)),
            pl.BlockSpec((1, 256), lambda t: (0, 0)),
            pl.BlockSpec((1, 256), lambda t: (0, 0)),
            pl.BlockSpec((256, 256), lambda t: (0, 0)),
            pl.BlockSpec((1, 256), lambda t: (0, 0)),
            pl.BlockSpec((1, 256), lambda t: (0, 0)),
            pl.BlockSpec((1, 256), lambda t: (0, 0)),
            pl.BlockSpec((256, num_tasks), lambda t: (0, 0)),
            pl.BlockSpec((1, num_tasks), lambda t: (0, 0)),
        ],
        out_specs=pl.BlockSpec((g_graphs, num_tasks), lambda t: (0, 0)),
        out_shape=jax.ShapeDtypeStruct((g_graphs, num_tasks), jnp.float32),
        scratch_shapes=[pltpu.VMEM((g_graphs, H), jnp.float32)],
    )(h4, batch3, w1, b1, g1, be1, w2, b2, g2, be2, w3, b3)


def _pad2(w, r, c):
    out = jnp.zeros((r, c), jnp.float32)
    return out.at[: w.shape[0], : w.shape[1]].set(w)


def _row(v):
    return v.reshape(1, -1)


def kernel(x, edge_index, batch_ind, params):
    g_graphs = 256
    num_tasks = params['clf']['W3'].shape[1]

    xp = jnp.zeros((N, HP), jnp.float32).at[:, : x.shape[1]].set(x)
    h4 = xp.reshape(N, NCH, F).transpose(1, 0, 2)
    src2 = edge_index[0].reshape(E // KB, KB)
    dst2 = edge_index[1].reshape(E // KB, KB)
    batch3 = batch_ind.reshape(NT, 1, RT)

    for i in range(5):
        p = params['gin'][i]
        first = (i == 0)
        w1 = _pad2(p['W1'], HP, H) if first else p['W1']

        m = _sc_segsum(h4.reshape(NCH * N, F), src2, dst2)
        h4 = _fused_layer(h4, m.reshape(NCH + 1, N, F), w1,
                          _row(p['b1']), _row(p['g1']), _row(p['be1']),
                          p['W2'], _row(p['b2']),
                          first, relu_out=(i < 4))

    c = params['clf']
    return _pool_classifier(
        h4, batch3,
        c['W1'], _row(c['b1']), _row(c['g1']), _row(c['be1']),
        c['W2'], _row(c['b2']), _row(c['g2']), _row(c['be2']),
        c['W3'], _row(c['b3']),
        num_tasks, g_graphs)


# speedup vs baseline: 1.0123x; 1.0123x over previous
import functools

import jax
import jax.numpy as jnp
from jax import lax
from jax.experimental import pallas as pl
from jax.experimental.pallas import tpu as pltpu
from jax.experimental.pallas import tpu_sc as plsc

N = 10000
E = 160000
H = 600
HP = 640
F = 128
NCH = 5
KB = 80
TILES = 16
BPT = (E // TILES) // KB
SLAB = N // TILES
ZR = 25
RT = 400
NT = N // RT
EPS = 1e-5


def _sc_segsum(hflat, src2, dst2):
    mesh = plsc.VectorSubcoreMesh(core_axis_name="c", subcore_axis_name="s")

    @functools.partial(
        pl.kernel,
        mesh=mesh,
        compiler_params=pltpu.CompilerParams(use_tc_tiling_on_sc=False),
        out_type=jax.ShapeDtypeStruct(((NCH + 1) * N, F), jnp.float32),
        scratch_types=[
            pltpu.VMEM((BPT, KB), jnp.int32),
            pltpu.VMEM((BPT, KB), jnp.int32),
            pltpu.VMEM((KB, F), jnp.float32),
            pltpu.VMEM((KB, F), jnp.float32),
            pltpu.VMEM((ZR, F), jnp.float32),
            pltpu.VMEM_SHARED((N, F), jnp.float32),
            pltpu.SemaphoreType.DMA,
            pltpu.SemaphoreType.DMA,
        ],
    )
    def k(h_hbm, src_hbm, dst_hbm, m_hbm, srcv, dstv, st0, st1, zbuf, acc,
          sem0, sem1):
        cc = lax.axis_index("c")
        ss = lax.axis_index("s")
        row0 = ss * BPT
        pltpu.sync_copy(src_hbm.at[pl.ds(row0, BPT)], srcv)
        pltpu.sync_copy(dst_hbm.at[pl.ds(row0, BPT)], dstv)

        def zrow(r, _):
            for kk in range(F // 16):
                zbuf[r, pl.ds(kk * 16, 16)] = jnp.zeros((16,), jnp.float32)
            return 0

        lax.fori_loop(0, ZR, zrow, 0)

        for j in range(3):
            c = 3 * cc + j
            if True:
                if j == 0:
                    delta = (3 * cc * N).astype(jnp.int32)
                elif j == 1:
                    delta = jnp.int32(N)
                else:
                    delta = jnp.where(cc == 0, N, -2 * N).astype(jnp.int32)

                def addrow(r, _):
                    for kk in range(KB // 16):
                        sl = pl.ds(kk * 16, 16)
                        srcv[r, sl] = srcv[r, sl] + delta
                    return 0

                lax.fori_loop(0, BPT, addrow, 0)

                for q in range(SLAB // ZR):
                    pltpu.sync_copy(
                        zbuf, acc.at[pl.ds(ss * SLAB + q * ZR, ZR)])
                plsc.subcore_barrier()

                if j < 2:
                    pltpu.async_copy(h_hbm.at[srcv.at[0]], st0, sem0)

                    def eblk(k2, _):
                        b0 = 2 * k2
                        b1 = b0 + 1
                        pltpu.async_copy(h_hbm.at[srcv.at[b1]], st1, sem1)
                        pltpu.make_async_copy(
                            h_hbm.at[srcv.at[b0]], st0, sem0).wait()
                        pltpu.sync_copy(st0, acc.at[dstv.at[b0]], add=True)

                        @pl.when(b0 + 2 < BPT)
                        def _():
                            pltpu.async_copy(
                                h_hbm.at[srcv.at[b0 + 2]], st0, sem0)

                        pltpu.make_async_copy(
                            h_hbm.at[srcv.at[b1]], st1, sem1).wait()
                        pltpu.sync_copy(st1, acc.at[dstv.at[b1]], add=True)
                        return 0

                    lax.fori_loop(0, (BPT - 1) // 2, eblk, 0)
                    pltpu.make_async_copy(
                        h_hbm.at[srcv.at[BPT - 1]], st0, sem0).wait()
                    pltpu.sync_copy(st0, acc.at[dstv.at[BPT - 1]], add=True)
                else:
                    lo = 62 * cc

                    pltpu.async_copy(h_hbm.at[srcv.at[lo]], st0, sem0)

                    def eblk2(k2, _):
                        b0 = lo + 2 * k2
                        b1 = b0 + 1
                        pltpu.async_copy(h_hbm.at[srcv.at[b1]], st1, sem1)
                        pltpu.make_async_copy(
                            h_hbm.at[srcv.at[b0]], st0, sem0).wait()
                        pltpu.sync_copy(st0, acc.at[dstv.at[b0]], add=True)

                        @pl.when(k2 + 1 < 31)
                        def _():
                            pltpu.async_copy(
                                h_hbm.at[srcv.at[b0 + 2]], st0, sem0)

                        pltpu.make_async_copy(
                            h_hbm.at[srcv.at[b1]], st1, sem1).wait()
                        pltpu.sync_copy(st1, acc.at[dstv.at[b1]], add=True)
                        return 0

                    lax.fori_loop(0, 31, eblk2, 0)

                    @pl.when(cc == 0)
                    def _():
                        pltpu.sync_copy(h_hbm.at[srcv.at[BPT - 1]], st0)
                        pltpu.sync_copy(
                            st0, acc.at[dstv.at[BPT - 1]], add=True)
                plsc.subcore_barrier()

                pltpu.sync_copy(
                    acc.at[pl.ds(ss * SLAB, SLAB)],
                    m_hbm.at[pl.ds(c * N + ss * SLAB, SLAB)],
                )
                plsc.subcore_barrier()

    return k(hflat, src2, dst2)


def _mm_stats(h4, m4, w1, b1, first_layer):
    kw = HP if first_layer else H

    def body(h_ref, m_ref, w_ref, b_ref, t_ref, s_ref, stats):
        i = pl.program_id(0)
        hm = h_ref[...]
        mm = m_ref[...]
        parts = []
        for cch in range(NCH):
            mc = mm[cch] + mm[NCH] if cch == 2 else mm[cch]
            parts.append(hm[cch] + mc)
        ut = jnp.concatenate(parts, axis=1)
        if first_layer:
            t = jnp.dot(ut, w_ref[...], preferred_element_type=jnp.float32)
            t = t[:, :H]
        else:
            t = jnp.dot(ut[:, :H], w_ref[...],
                        preferred_element_type=jnp.float32)
        t = t + b_ref[...]
        t_ref[...] = t

        @pl.when(i == 0)
        def _():
            stats[...] = jnp.zeros_like(stats)

        stats[0:1] = stats[0:1] + jnp.sum(t, axis=0, keepdims=True)
        stats[1:2] = stats[1:2] + jnp.sum(t * t, axis=0, keepdims=True)
        s_ref[...] = stats[...]

    return pl.pallas_call(
        body,
        grid=(NT,),
        in_specs=[
            pl.BlockSpec((NCH, RT, F), lambda i: (0, i, 0)),
            pl.BlockSpec((NCH + 1, RT, F), lambda i: (0, i, 0)),
            pl.BlockSpec((kw, H), lambda i: (0, 0)),
            pl.BlockSpec((1, H), lambda i: (0, 0)),
        ],
        out_specs=[
            pl.BlockSpec((RT, H), lambda i: (i, 0)),
            pl.BlockSpec((8, H), lambda i: (0, 0)),
        ],
        out_shape=[
            jax.ShapeDtypeStruct((N, H), jnp.float32),
            jax.ShapeDtypeStruct((8, H), jnp.float32),
        ],
        scratch_shapes=[pltpu.VMEM((8, H), jnp.float32)],
    )(h4, m4, w1, b1)


def _bn_relu_mm(t, stats, g1, be1, w2, b2, relu_out):
    def body(t_ref, s_ref, g_ref, be_ref, w_ref, b_ref, out_ref):
        mean = s_ref[0:1] / N
        var = s_ref[1:2] / N - mean * mean
        scale = g_ref[...] * lax.rsqrt(var + EPS)
        shift = be_ref[...] - mean * scale
        z = jnp.maximum(t_ref[...] * scale + shift, 0.0)
        o = jnp.dot(z, w_ref[...], preferred_element_type=jnp.float32)
        o = o + b_ref[...]
        if relu_out:
            o = jnp.maximum(o, 0.0)
        o = jnp.concatenate([o, jnp.zeros((RT, HP - H), jnp.float32)], axis=1)
        for cch in range(NCH):
            out_ref[cch] = o[:, cch * F:(cch + 1) * F]

    return pl.pallas_call(
        body,
        grid=(NT,),
        in_specs=[
            pl.BlockSpec((RT, H), lambda i: (i, 0)),
            pl.BlockSpec((8, H), lambda i: (0, 0)),
            pl.BlockSpec((1, H), lambda i: (0, 0)),
            pl.BlockSpec((1, H), lambda i: (0, 0)),
            pl.BlockSpec((H, H), lambda i: (0, 0)),
            pl.BlockSpec((1, H), lambda i: (0, 0)),
        ],
        out_specs=pl.BlockSpec((NCH, RT, F), lambda i: (0, i, 0)),
        out_shape=jax.ShapeDtypeStruct((NCH, N, F), jnp.float32),
    )(t, stats, g1, be1, w2, b2)


def _pool_classifier(h4, batch3, w1, b1, g1, be1, w2, b2, g2, be2, w3, b3,
                     num_tasks, g_graphs):
    def bn(y, g, be):
        mu = jnp.mean(y, axis=0, keepdims=True)
        var = jnp.mean(y * y, axis=0, keepdims=True) - mu * mu
        return (y - mu) * lax.rsqrt(var + EPS) * g + be

    def body(h_ref, b_ref, w1_ref, b1_ref, g1_ref, be1_ref, w2_ref, b2_ref,
             g2_ref, be2_ref, w3_ref, b3_ref, out_ref, feats):
        t = pl.program_id(0)

        @pl.when(t == 0)
        def _():
            feats[...] = jnp.zeros_like(feats)

        @pl.when(t < NT)
        def _():
            ht = h_ref[...].transpose(1, 0, 2).reshape(RT, HP)[:, :H]
            gids = b_ref[0, 0, :]
            rows = lax.broadcasted_iota(jnp.int32, (g_graphs, RT), 0)
            oh = (rows == gids[None, :]).astype(jnp.float32)
            feats[...] = feats[...] + jnp.dot(
                oh, ht, preferred_element_type=jnp.float32,
                precision=lax.Precision.HIGHEST)

        @pl.when(t == NT)
        def _():
            f = feats[...]
            y = jnp.dot(f, w1_ref[...], preferred_element_type=jnp.float32)
            y = jnp.maximum(bn(y + b1_ref[...], g1_ref[...], be1_ref[...]), 0.0)
            y = jnp.dot(y, w2_ref[...], preferred_element_type=jnp.float32)
            y = jnp.maximum(bn(y + b2_ref[...], g2_ref[...], be2_ref[...]), 0.0)
            y = jnp.dot(y, w3_ref[...], preferred_element_type=jnp.float32)
            out_ref[...] = y + b3_ref[...]

    def clamp(t):
        return jnp.minimum(t, NT - 1)

    return pl.pallas_call(
        body,
        grid=(NT + 1,),
        in_specs=[
            pl.BlockSpec((NCH, RT, F), lambda t: (0, clamp(t), 0)),
            pl.BlockSpec((1, 1, RT), lambda t: (clamp(t), 0, 0)),
            pl.BlockSpec((H, 256), lambda t: (0, 0)),
            pl.BlockSpec((1, 256), lambda t: (0, 0)),
            pl.BlockSpec((1, 256), lambda t: (0, 0)),
            pl.BlockSpec((1, 256), lambda t: (0, 0)),
            pl.BlockSpec((256, 256), lambda t: (0, 0)),
            pl.BlockSpec((1, 256), lambda t: (0, 0)),
            pl.BlockSpec((1, 256), lambda t: (0, 0)),
            pl.BlockSpec((1, 256), lambda t: (0, 0)),
            pl.BlockSpec((256, num_tasks), lambda t: (0, 0)),
            pl.BlockSpec((1, num_tasks), lambda t: (0, 0)),
        ],
        out_specs=pl.BlockSpec((g_graphs, num_tasks), lambda t: (0, 0)),
        out_shape=jax.ShapeDtypeStruct((g_graphs, num_tasks), jnp.float32),
        scratch_shapes=[pltpu.VMEM((g_graphs, H), jnp.float32)],
    )(h4, batch3, w1, b1, g1, be1, w2, b2, g2, be2, w3, b3)


def _pad2(w, r, c):
    out = jnp.zeros((r, c), jnp.float32)
    return out.at[: w.shape[0], : w.shape[1]].set(w)


def _row(v):
    return v.reshape(1, -1)


def kernel(x, edge_index, batch_ind, params):
    g_graphs = 256
    num_tasks = params['clf']['W3'].shape[1]

    xp = jnp.zeros((N, HP), jnp.float32).at[:, : x.shape[1]].set(x)
    h4 = xp.reshape(N, NCH, F).transpose(1, 0, 2)
    src2 = edge_index[0].reshape(E // KB, KB)
    dst2 = edge_index[1].reshape(E // KB, KB)
    batch3 = batch_ind.reshape(NT, 1, RT)

    for i in range(5):
        p = params['gin'][i]
        first = (i == 0)
        w1 = _pad2(p['W1'], HP, H) if first else p['W1']

        m = _sc_segsum(h4.reshape(NCH * N, F), src2, dst2)
        t, stats = _mm_stats(h4, m.reshape(NCH + 1, N, F), w1,
                             _row(p['b1']), first)
        h4 = _bn_relu_mm(t, stats, _row(p['g1']), _row(p['be1']),
                         p['W2'], _row(p['b2']),
                         relu_out=(i < 4))

    c = params['clf']
    return _pool_classifier(
        h4, batch3,
        c['W1'], _row(c['b1']), _row(c['g1']), _row(c['be1']),
        c['W2'], _row(c['b2']), _row(c['g2']), _row(c['be2']),
        c['W3'], _row(c['b3']),
        num_tasks, g_graphs)
